# baseline (speedup 1.0000x reference)
import jax
import jax.numpy as jnp
from jax.experimental import pallas as pl
from jax.experimental.pallas import tpu as pltpu

_BBLK = 8
_HWBLK = 144


def _router_body(x_ref, w1_ref, b1_ref, w2_ref, b2_ref, out_ref, acc):
    j = pl.program_id(1)
    nj = pl.num_programs(1)
    part = jnp.sum(x_ref[...], axis=1)

    @pl.when(j == 0)
    def _init():
        acc[...] = part

    @pl.when(j > 0)
    def _accum():
        acc[...] += part

    @pl.when(j == nj - 1)
    def _finish():
        pooled = acc[...] * (1.0 / (nj * _HWBLK))
        h = pooled @ w1_ref[...] + b1_ref[...]
        h = 0.5 * h * (1.0 + jax.lax.erf(h * (2.0 ** -0.5)))
        logits = h @ w2_ref[...] + b2_ref[...]
        m = jnp.max(logits, axis=-1, keepdims=True)
        e = jnp.exp(logits - m)
        out_ref[...] = e / jnp.sum(e, axis=-1, keepdims=True)


def kernel(x, W1, b1, W2, b2):
    B, C, H, W = x.shape
    hw = H * W
    xt = jnp.transpose(x, (0, 2, 3, 1)).reshape(B, hw, C)
    grid = (B // _BBLK, hw // _HWBLK)
    return pl.pallas_call(
        _router_body,
        grid=grid,
        in_specs=[
            pl.BlockSpec((_BBLK, _HWBLK, C), lambda i, j: (i, j, 0)),
            pl.BlockSpec((C, W1.shape[1]), lambda i, j: (0, 0)),
            pl.BlockSpec((W1.shape[1],), lambda i, j: (0,)),
            pl.BlockSpec((W1.shape[1], W2.shape[1]), lambda i, j: (0, 0)),
            pl.BlockSpec((W2.shape[1],), lambda i, j: (0,)),
        ],
        out_specs=pl.BlockSpec((_BBLK, W2.shape[1]), lambda i, j: (i, 0)),
        out_shape=jax.ShapeDtypeStruct((B, W2.shape[1]), jnp.float32),
        scratch_shapes=[pltpu.VMEM((_BBLK, C), jnp.float32)],
    )(xt, W1, b1, W2, b2)


# speedup vs baseline: 1.1831x; 1.1831x over previous
import jax
import jax.numpy as jnp
from jax.experimental import pallas as pl
from jax.experimental.pallas import tpu as pltpu

_BBLK = 8


def _router_body(x_ref, w1_ref, b1_ref, w2_ref, b2_ref, out_ref):
    hw = x_ref.shape[1]
    pooled = jnp.sum(x_ref[...], axis=1) * (1.0 / hw)
    h = pooled @ w1_ref[...] + b1_ref[...]
    h = 0.5 * h * (1.0 + jax.lax.erf(h * (2.0 ** -0.5)))
    logits = h @ w2_ref[...] + b2_ref[...]
    m = jnp.max(logits, axis=-1, keepdims=True)
    e = jnp.exp(logits - m)
    out_ref[...] = e / jnp.sum(e, axis=-1, keepdims=True)


def kernel(x, W1, b1, W2, b2):
    B, C, H, W = x.shape
    hw = H * W
    xt = jnp.transpose(x, (0, 2, 3, 1)).reshape(B, hw, C)
    grid = (B // _BBLK,)
    return pl.pallas_call(
        _router_body,
        grid=grid,
        in_specs=[
            pl.BlockSpec((_BBLK, hw, C), lambda i: (i, 0, 0)),
            pl.BlockSpec((C, W1.shape[1]), lambda i: (0, 0)),
            pl.BlockSpec((W1.shape[1],), lambda i: (0,)),
            pl.BlockSpec((W1.shape[1], W2.shape[1]), lambda i: (0, 0)),
            pl.BlockSpec((W2.shape[1],), lambda i: (0,)),
        ],
        out_specs=pl.BlockSpec((_BBLK, W2.shape[1]), lambda i: (i, 0)),
        out_shape=jax.ShapeDtypeStruct((B, W2.shape[1]), jnp.float32),
    )(xt, W1, b1, W2, b2)


# speedup vs baseline: 1.2316x; 1.0410x over previous
import jax
import jax.numpy as jnp
from jax.experimental import pallas as pl
from jax.experimental.pallas import tpu as pltpu

_BBLK = 4


def _router_body(x_ref, w1_ref, b1_ref, w2_ref, b2_ref, out_ref):
    hw = x_ref.shape[1]
    pooled = jnp.sum(x_ref[...], axis=1) * (1.0 / hw)
    h = pooled @ w1_ref[...] + b1_ref[...]
    h = 0.5 * h * (1.0 + jax.lax.erf(h * (2.0 ** -0.5)))
    logits = h @ w2_ref[...] + b2_ref[...]
    m = jnp.max(logits, axis=-1, keepdims=True)
    e = jnp.exp(logits - m)
    out_ref[0, :, :] = e / jnp.sum(e, axis=-1, keepdims=True)


def kernel(x, W1, b1, W2, b2):
    B, C, H, W = x.shape
    hw = H * W
    E = W2.shape[1]
    xt = jnp.transpose(x, (0, 2, 3, 1)).reshape(B, hw, C)
    grid = (B // _BBLK,)
    out = pl.pallas_call(
        _router_body,
        grid=grid,
        in_specs=[
            pl.BlockSpec((_BBLK, hw, C), lambda i: (i, 0, 0)),
            pl.BlockSpec((C, W1.shape[1]), lambda i: (0, 0)),
            pl.BlockSpec((W1.shape[1],), lambda i: (0,)),
            pl.BlockSpec((W1.shape[1], E), lambda i: (0, 0)),
            pl.BlockSpec((E,), lambda i: (0,)),
        ],
        out_specs=pl.BlockSpec((1, _BBLK, E), lambda i: (i, 0, 0)),
        out_shape=jax.ShapeDtypeStruct((B // _BBLK, _BBLK, E), jnp.float32),
    )(xt, W1, b1, W2, b2)
    return out.reshape(B, E)
